# baseline (speedup 1.0000x reference)
import functools

import jax
import jax.numpy as jnp
from jax import lax
from jax.experimental import pallas as pl
from jax.experimental.pallas import tpu as pltpu
from jax.experimental.pallas import tpu_sc as plsc

EPS = 1e-5

NUM_SC = 2
NUM_SUBCORES = 16
NUM_WORKERS = NUM_SC * NUM_SUBCORES
LANES = 16

CHUNK = 80
K = 2


def _cst_send_body(x_ref, w1_ref, b1_ref, w2_ref, g1_ref, be1_ref, o_ref):
    x = x_ref[...]
    h = lax.dot_general(x, w1_ref[...], (((1,), (1,)), ((), ())),
                        preferred_element_type=jnp.float32)
    h = jnp.maximum(h + b1_ref[...], 0.0)
    m = lax.dot_general(h, w2_ref[...], (((1,), (1,)), ((), ())),
                        preferred_element_type=jnp.float32)
    mu = jnp.mean(m, axis=-1, keepdims=True)
    v = jnp.mean((m - mu) ** 2, axis=-1, keepdims=True)
    o_ref[...] = (m - mu) / jnp.sqrt(v + EPS) * g1_ref[...] + be1_ref[...]


def _cst_send(r_cst, W1, b1, W2, g1, be1, *, block_rows=1000):
    nc, h = r_cst.shape
    h4 = W2.shape[0]
    grid = nc // block_rows
    return pl.pallas_call(
        _cst_send_body,
        grid=(grid,),
        in_specs=[
            pl.BlockSpec((block_rows, h), lambda i: (i, 0)),
            pl.BlockSpec((h, h), lambda i: (0, 0)),
            pl.BlockSpec((1, h), lambda i: (0, 0)),
            pl.BlockSpec((h4, h), lambda i: (0, 0)),
            pl.BlockSpec((1, h4), lambda i: (0, 0)),
            pl.BlockSpec((1, h4), lambda i: (0, 0)),
        ],
        out_specs=pl.BlockSpec((block_rows, h4), lambda i: (i, 0)),
        out_shape=jax.ShapeDtypeStruct((nc, h4), jnp.float32),
    )(r_cst, W1, b1.reshape(1, h), W2, g1.reshape(1, h4), be1.reshape(1, h4))


def _edge_agg(m_cst, ce0, ce1, le, pe, nvp):
    h = m_cst.shape[1]
    e = ce0.shape[0]
    ept = e // NUM_WORKERS
    rows = ept // CHUNK
    groups = rows // K
    tail = rows - K * groups
    rows_per_tile = nvp // NUM_SUBCORES
    zrep = rows_per_tile // CHUNK
    mesh = plsc.VectorSubcoreMesh(
        core_axis_name="c", subcore_axis_name="s",
        num_cores=NUM_SC, num_subcores=NUM_SUBCORES)

    @functools.partial(
        pl.kernel,
        out_type=jax.ShapeDtypeStruct((NUM_SC, nvp, h), jnp.float32),
        mesh=mesh,
        scratch_types=[
            [pltpu.VMEM((4, CHUNK), jnp.int32) for _ in range(K)],
            [pltpu.VMEM((CHUNK, h), jnp.float32) for _ in range(K)],
            pltpu.VMEM_SHARED((nvp, h), jnp.float32),
            [pltpu.SemaphoreType.DMA for _ in range(K)],
            [pltpu.SemaphoreType.DMA for _ in range(K)],
        ],
    )
    def edge_kernel(m_ref, ce0_ref, ce1_ref, le_ref, pe_ref, out_ref,
                    ibuf, msg, acc_ref, gsem, isem):
        c = lax.axis_index("c")
        s = lax.axis_index("s")
        wid = c * NUM_SUBCORES + s
        r0 = s * rows_per_tile
        base = wid * ept

        def zbody(t, carry):
            msg[0][t // (h // LANES),
                   pl.ds((t % (h // LANES)) * LANES, LANES)] = jnp.zeros(
                (LANES,), jnp.float32)
            return carry
        lax.fori_loop(0, CHUNK * h // LANES, zbody, 0)
        for z in range(zrep):
            pltpu.sync_copy(msg[0], acc_ref.at[pl.ds(r0 + z * CHUNK, CHUNK)])
        plsc.subcore_barrier()

        def load_and_gather(r, b):
            off = base + r * CHUNK
            ds = [
                pltpu.async_copy(ce0_ref.at[pl.ds(off, CHUNK)],
                                 ibuf[b].at[0], isem[b]),
                pltpu.async_copy(le_ref.at[pl.ds(off, CHUNK)],
                                 ibuf[b].at[1], isem[b]),
                pltpu.async_copy(pe_ref.at[pl.ds(off, CHUNK)],
                                 ibuf[b].at[2], isem[b]),
                pltpu.async_copy(ce1_ref.at[pl.ds(off, CHUNK)],
                                 ibuf[b].at[3], isem[b]),
            ]
            for d in ds:
                d.wait()
            for cc in range(CHUNK // LANES):
                sl = pl.ds(cc * LANES, LANES)
                ibuf[b][0, sl] = (ibuf[b][0, sl] * 4 + ibuf[b][1, sl] * 2
                                  + ibuf[b][2, sl])
            return pltpu.async_copy(m_ref.at[ibuf[b].at[0]], msg[b], gsem[b])

        def gbody(g, carry):
            descs = [load_and_gather(g * K + b, b) for b in range(K)]
            for b in range(K):
                descs[b].wait()
                pltpu.sync_copy(msg[b], acc_ref.at[ibuf[b].at[3]], add=True)
            return carry

        lax.fori_loop(0, groups, gbody, 0)
        for t in range(tail):
            d = load_and_gather(K * groups + t, 0)
            d.wait()
            pltpu.sync_copy(msg[0], acc_ref.at[ibuf[0].at[3]], add=True)

        plsc.subcore_barrier()
        pltpu.sync_copy(acc_ref.at[pl.ds(r0, rows_per_tile)],
                        out_ref.at[c, pl.ds(r0, rows_per_tile)])

    return edge_kernel(m_cst, ce0, ce1, le, pe)


def _val_rec_body(x_ref, p_ref, w3_ref, b3_ref, w4_ref, g2_ref, be2_ref, o_ref):
    x = x_ref[...]
    z = x + p_ref[0] + p_ref[1]
    u = lax.dot_general(z, w3_ref[...], (((1,), (1,)), ((), ())),
                        preferred_element_type=jnp.float32)
    u = jnp.maximum(u + b3_ref[...], 0.0)
    m = lax.dot_general(u, w4_ref[...], (((1,), (1,)), ((), ())),
                        preferred_element_type=jnp.float32)
    mu = jnp.mean(m, axis=-1, keepdims=True)
    v = jnp.mean((m - mu) ** 2, axis=-1, keepdims=True)
    o_ref[...] = (m - mu) / jnp.sqrt(v + EPS) * g2_ref[...] + be2_ref[...] + x


def _val_rec(x_val, partials, W3, b3, W4, g2, be2, *, block_rows=1000):
    nv, h = x_val.shape
    grid = nv // block_rows
    return pl.pallas_call(
        _val_rec_body,
        grid=(grid,),
        in_specs=[
            pl.BlockSpec((block_rows, h), lambda i: (i, 0)),
            pl.BlockSpec((NUM_SC, block_rows, h), lambda i: (0, i, 0)),
            pl.BlockSpec((h, h), lambda i: (0, 0)),
            pl.BlockSpec((1, h), lambda i: (0, 0)),
            pl.BlockSpec((h, h), lambda i: (0, 0)),
            pl.BlockSpec((1, h), lambda i: (0, 0)),
            pl.BlockSpec((1, h), lambda i: (0, 0)),
        ],
        out_specs=pl.BlockSpec((block_rows, h), lambda i: (i, 0)),
        out_shape=jax.ShapeDtypeStruct((nv, h), jnp.float32),
    )(x_val, partials, W3, b3.reshape(1, h), W4, g2.reshape(1, h), be2.reshape(1, h))


def kernel(x_val, r_cst, cst_edges, LE, PE, num_cst, num_val, W1, b1, W2,
           g1, be1, W3, b3, W4, g2, be2):
    nc, h = r_cst.shape
    nv = x_val.shape[0]
    m_cst = _cst_send(r_cst, W1, b1, W2, g1, be1)
    m_cst = m_cst.reshape(4 * nc, h)
    quantum = CHUNK * NUM_SUBCORES
    nvp = ((nv + quantum - 1) // quantum) * quantum
    partials = _edge_agg(m_cst, cst_edges[0], cst_edges[1], LE, PE, nvp)
    return _val_rec(x_val, partials, W3, b3, W4, g2, be2)


# speedup vs baseline: 1.0393x; 1.0393x over previous
import functools

import jax
import jax.numpy as jnp
from jax import lax
from jax.experimental import pallas as pl
from jax.experimental.pallas import tpu as pltpu
from jax.experimental.pallas import tpu_sc as plsc

EPS = 1e-5

NUM_SC = 2
NUM_SUBCORES = 16
NUM_WORKERS = NUM_SC * NUM_SUBCORES
LANES = 16

CHUNK = 80
K = 2


def _cst_send_body(x_ref, w1_ref, b1_ref, w2_ref, g1_ref, be1_ref, o_ref):
    x = x_ref[...]
    h = lax.dot_general(x, w1_ref[...], (((1,), (1,)), ((), ())),
                        preferred_element_type=jnp.float32)
    h = jnp.maximum(h + b1_ref[...], 0.0)
    m = lax.dot_general(h, w2_ref[...], (((1,), (1,)), ((), ())),
                        preferred_element_type=jnp.float32)
    mu = jnp.mean(m, axis=-1, keepdims=True)
    v = jnp.mean((m - mu) ** 2, axis=-1, keepdims=True)
    o_ref[...] = (m - mu) / jnp.sqrt(v + EPS) * g1_ref[...] + be1_ref[...]


def _cst_send(r_cst, W1, b1, W2, g1, be1, *, block_rows=1000):
    nc, h = r_cst.shape
    h4 = W2.shape[0]
    grid = nc // block_rows
    return pl.pallas_call(
        _cst_send_body,
        grid=(grid,),
        in_specs=[
            pl.BlockSpec((block_rows, h), lambda i: (i, 0)),
            pl.BlockSpec((h, h), lambda i: (0, 0)),
            pl.BlockSpec((1, h), lambda i: (0, 0)),
            pl.BlockSpec((h4, h), lambda i: (0, 0)),
            pl.BlockSpec((1, h4), lambda i: (0, 0)),
            pl.BlockSpec((1, h4), lambda i: (0, 0)),
        ],
        out_specs=pl.BlockSpec((block_rows, h4), lambda i: (i, 0)),
        out_shape=jax.ShapeDtypeStruct((nc, h4), jnp.float32),
    )(r_cst, W1, b1.reshape(1, h), W2, g1.reshape(1, h4), be1.reshape(1, h4))


def _edge_agg(m_cst, ce_flat, le, pe, nvp):
    h = m_cst.shape[1]
    e = le.shape[0]
    ept = e // NUM_WORKERS
    rows = ept // CHUNK
    groups = rows // K
    tail = rows - K * groups
    rows_per_tile = nvp // NUM_SUBCORES
    zrep = rows_per_tile // CHUNK
    mesh = plsc.VectorSubcoreMesh(
        core_axis_name="c", subcore_axis_name="s",
        num_cores=NUM_SC, num_subcores=NUM_SUBCORES)

    @functools.partial(
        pl.kernel,
        out_type=jax.ShapeDtypeStruct((NUM_SC, nvp, h), jnp.float32),
        mesh=mesh,
        scratch_types=[
            [pltpu.VMEM((4, CHUNK), jnp.int32) for _ in range(K)],
            [pltpu.VMEM((CHUNK, h), jnp.float32) for _ in range(K)],
            pltpu.VMEM_SHARED((nvp, h), jnp.float32),
            [pltpu.SemaphoreType.DMA for _ in range(K)],
            [pltpu.SemaphoreType.DMA for _ in range(K)],
        ],
    )
    def edge_kernel(m_ref, ce_ref, le_ref, pe_ref, out_ref,
                    ibuf, msg, acc_ref, gsem, isem):
        c = lax.axis_index("c")
        s = lax.axis_index("s")
        wid = c * NUM_SUBCORES + s
        r0 = s * rows_per_tile
        base = wid * ept

        def zbody(t, carry):
            msg[0][t // (h // LANES),
                   pl.ds((t % (h // LANES)) * LANES, LANES)] = jnp.zeros(
                (LANES,), jnp.float32)
            return carry
        lax.fori_loop(0, CHUNK * h // LANES, zbody, 0)
        for z in range(zrep):
            pltpu.sync_copy(msg[0], acc_ref.at[pl.ds(r0 + z * CHUNK, CHUNK)])
        plsc.subcore_barrier()

        def load_and_gather(r, b):
            off = base + r * CHUNK
            ds = [
                pltpu.async_copy(ce_ref.at[pl.ds(off, CHUNK)],
                                 ibuf[b].at[0], isem[b]),
                pltpu.async_copy(le_ref.at[pl.ds(off, CHUNK)],
                                 ibuf[b].at[1], isem[b]),
                pltpu.async_copy(pe_ref.at[pl.ds(off, CHUNK)],
                                 ibuf[b].at[2], isem[b]),
                pltpu.async_copy(ce_ref.at[pl.ds(e + off, CHUNK)],
                                 ibuf[b].at[3], isem[b]),
            ]
            for d in ds:
                d.wait()
            for cc in range(CHUNK // LANES):
                sl = pl.ds(cc * LANES, LANES)
                ibuf[b][0, sl] = (ibuf[b][0, sl] * 4 + ibuf[b][1, sl] * 2
                                  + ibuf[b][2, sl])
            return pltpu.async_copy(m_ref.at[ibuf[b].at[0]], msg[b], gsem[b])

        def gbody(g, carry):
            descs = [load_and_gather(g * K + b, b) for b in range(K)]
            for b in range(K):
                descs[b].wait()
                pltpu.sync_copy(msg[b], acc_ref.at[ibuf[b].at[3]], add=True)
            return carry

        lax.fori_loop(0, groups, gbody, 0)
        for t in range(tail):
            d = load_and_gather(K * groups + t, 0)
            d.wait()
            pltpu.sync_copy(msg[0], acc_ref.at[ibuf[0].at[3]], add=True)

        plsc.subcore_barrier()
        pltpu.sync_copy(acc_ref.at[pl.ds(r0, rows_per_tile)],
                        out_ref.at[c, pl.ds(r0, rows_per_tile)])

    return edge_kernel(m_cst, ce_flat, le, pe)


def _val_rec_body(x_ref, p_ref, w3_ref, b3_ref, w4_ref, g2_ref, be2_ref, o_ref):
    x = x_ref[...]
    z = x + p_ref[0] + p_ref[1]
    u = lax.dot_general(z, w3_ref[...], (((1,), (1,)), ((), ())),
                        preferred_element_type=jnp.float32)
    u = jnp.maximum(u + b3_ref[...], 0.0)
    m = lax.dot_general(u, w4_ref[...], (((1,), (1,)), ((), ())),
                        preferred_element_type=jnp.float32)
    mu = jnp.mean(m, axis=-1, keepdims=True)
    v = jnp.mean((m - mu) ** 2, axis=-1, keepdims=True)
    o_ref[...] = (m - mu) / jnp.sqrt(v + EPS) * g2_ref[...] + be2_ref[...] + x


def _val_rec(x_val, partials, W3, b3, W4, g2, be2, *, block_rows=1000):
    nv, h = x_val.shape
    grid = nv // block_rows
    return pl.pallas_call(
        _val_rec_body,
        grid=(grid,),
        in_specs=[
            pl.BlockSpec((block_rows, h), lambda i: (i, 0)),
            pl.BlockSpec((NUM_SC, block_rows, h), lambda i: (0, i, 0)),
            pl.BlockSpec((h, h), lambda i: (0, 0)),
            pl.BlockSpec((1, h), lambda i: (0, 0)),
            pl.BlockSpec((h, h), lambda i: (0, 0)),
            pl.BlockSpec((1, h), lambda i: (0, 0)),
            pl.BlockSpec((1, h), lambda i: (0, 0)),
        ],
        out_specs=pl.BlockSpec((block_rows, h), lambda i: (i, 0)),
        out_shape=jax.ShapeDtypeStruct((nv, h), jnp.float32),
    )(x_val, partials, W3, b3.reshape(1, h), W4, g2.reshape(1, h), be2.reshape(1, h))


def kernel(x_val, r_cst, cst_edges, LE, PE, num_cst, num_val, W1, b1, W2,
           g1, be1, W3, b3, W4, g2, be2):
    nc, h = r_cst.shape
    nv = x_val.shape[0]
    m_cst = _cst_send(r_cst, W1, b1, W2, g1, be1)
    m_cst = m_cst.reshape(4 * nc, h)
    quantum = CHUNK * NUM_SUBCORES
    nvp = ((nv + quantum - 1) // quantum) * quantum
    partials = _edge_agg(m_cst, cst_edges.reshape(-1), LE, PE, nvp)
    return _val_rec(x_val, partials, W3, b3, W4, g2, be2)


# speedup vs baseline: 1.1739x; 1.1295x over previous
import functools

import jax
import jax.numpy as jnp
from jax import lax
from jax.experimental import pallas as pl
from jax.experimental.pallas import tpu as pltpu
from jax.experimental.pallas import tpu_sc as plsc

EPS = 1e-5

NUM_SC = 2
NUM_SUBCORES = 16
NUM_WORKERS = NUM_SC * NUM_SUBCORES
LANES = 16

CHUNK = 80
K = 2


def _cst_send_body(x_ref, w1_ref, b1_ref, w2_ref, g1_ref, be1_ref, o_ref):
    x = x_ref[...]
    h = lax.dot_general(x, w1_ref[...], (((1,), (1,)), ((), ())),
                        preferred_element_type=jnp.float32)
    h = jnp.maximum(h + b1_ref[...], 0.0)
    m = lax.dot_general(h, w2_ref[...], (((1,), (1,)), ((), ())),
                        preferred_element_type=jnp.float32)
    mu = jnp.mean(m, axis=-1, keepdims=True)
    v = jnp.mean((m - mu) ** 2, axis=-1, keepdims=True)
    o_ref[...] = (m - mu) / jnp.sqrt(v + EPS) * g1_ref[...] + be1_ref[...]


def _cst_send(r_cst, W1, b1, W2, g1, be1, *, block_rows=1000):
    nc, h = r_cst.shape
    h4 = W2.shape[0]
    grid = nc // block_rows
    return pl.pallas_call(
        _cst_send_body,
        grid=(grid,),
        in_specs=[
            pl.BlockSpec((block_rows, h), lambda i: (i, 0)),
            pl.BlockSpec((h, h), lambda i: (0, 0)),
            pl.BlockSpec((1, h), lambda i: (0, 0)),
            pl.BlockSpec((h4, h), lambda i: (0, 0)),
            pl.BlockSpec((1, h4), lambda i: (0, 0)),
            pl.BlockSpec((1, h4), lambda i: (0, 0)),
        ],
        out_specs=pl.BlockSpec((block_rows, h4), lambda i: (i, 0)),
        out_shape=jax.ShapeDtypeStruct((nc, h4), jnp.float32),
    )(r_cst, W1, b1.reshape(1, h), W2, g1.reshape(1, h4), be1.reshape(1, h4))


def _edge_agg(m_cst, ce_flat, le, pe, nvp):
    h = m_cst.shape[1]
    e = le.shape[0]
    ept = e // NUM_WORKERS
    rows = ept // CHUNK
    groups = rows // K
    tail = rows - K * groups
    rows_per_tile = nvp // NUM_SUBCORES
    zrep = rows_per_tile // CHUNK
    mesh = plsc.VectorSubcoreMesh(
        core_axis_name="c", subcore_axis_name="s",
        num_cores=NUM_SC, num_subcores=NUM_SUBCORES)

    @functools.partial(
        pl.kernel,
        out_type=jax.ShapeDtypeStruct((NUM_SC, nvp, h), jnp.float32),
        mesh=mesh,
        scratch_types=[
            [pltpu.VMEM((4, CHUNK), jnp.int32) for _ in range(K)],
            [pltpu.VMEM((CHUNK, h), jnp.float32) for _ in range(K)],
            pltpu.VMEM_SHARED((nvp, h), jnp.float32),
            [pltpu.SemaphoreType.DMA for _ in range(K)],
            [pltpu.SemaphoreType.DMA for _ in range(K)],
        ],
    )
    def edge_kernel(m_ref, ce_ref, le_ref, pe_ref, out_ref,
                    ibuf, msg, acc_ref, gsem, isem):
        c = lax.axis_index("c")
        s = lax.axis_index("s")
        wid = c * NUM_SUBCORES + s
        r0 = s * rows_per_tile
        base = wid * ept

        def zbody(t, carry):
            msg[0][t // (h // LANES),
                   pl.ds((t % (h // LANES)) * LANES, LANES)] = jnp.zeros(
                (LANES,), jnp.float32)
            return carry
        lax.fori_loop(0, CHUNK * h // LANES, zbody, 0)
        for z in range(zrep):
            pltpu.sync_copy(msg[0], acc_ref.at[pl.ds(r0 + z * CHUNK, CHUNK)])
        plsc.subcore_barrier()

        def load_and_gather(r, b):
            off = base + r * CHUNK
            ds = [
                pltpu.async_copy(ce_ref.at[pl.ds(off, CHUNK)],
                                 ibuf[b].at[0], isem[b]),
                pltpu.async_copy(le_ref.at[pl.ds(off, CHUNK)],
                                 ibuf[b].at[1], isem[b]),
                pltpu.async_copy(pe_ref.at[pl.ds(off, CHUNK)],
                                 ibuf[b].at[2], isem[b]),
                pltpu.async_copy(ce_ref.at[pl.ds(e + off, CHUNK)],
                                 ibuf[b].at[3], isem[b]),
            ]
            for d in ds:
                d.wait()
            for cc in range(CHUNK // LANES):
                sl = pl.ds(cc * LANES, LANES)
                ibuf[b][0, sl] = (ibuf[b][0, sl] * 4 + ibuf[b][1, sl] * 2
                                  + ibuf[b][2, sl])
            return pltpu.async_copy(m_ref.at[ibuf[b].at[0]], msg[b], gsem[b])

        for b in range(K):
            load_and_gather(b, b)

        def ring_step(i, b):
            pltpu.make_async_copy(m_ref.at[ibuf[b].at[0]], msg[b],
                                  gsem[b]).wait()
            pltpu.sync_copy(msg[b], acc_ref.at[ibuf[b].at[3]], add=True)

            @pl.when(i + K < rows)
            def _():
                load_and_gather(i + K, b)

        def rbody(i, carry):
            for b in range(K):
                @pl.when(i % K == b)
                def _():
                    ring_step(i, b)
            return carry

        lax.fori_loop(0, rows, rbody, 0)

        plsc.subcore_barrier()
        pltpu.sync_copy(acc_ref.at[pl.ds(r0, rows_per_tile)],
                        out_ref.at[c, pl.ds(r0, rows_per_tile)])

    return edge_kernel(m_cst, ce_flat, le, pe)


def _val_rec_body(x_ref, p_ref, w3_ref, b3_ref, w4_ref, g2_ref, be2_ref, o_ref):
    x = x_ref[...]
    z = x + p_ref[0] + p_ref[1]
    u = lax.dot_general(z, w3_ref[...], (((1,), (1,)), ((), ())),
                        preferred_element_type=jnp.float32)
    u = jnp.maximum(u + b3_ref[...], 0.0)
    m = lax.dot_general(u, w4_ref[...], (((1,), (1,)), ((), ())),
                        preferred_element_type=jnp.float32)
    mu = jnp.mean(m, axis=-1, keepdims=True)
    v = jnp.mean((m - mu) ** 2, axis=-1, keepdims=True)
    o_ref[...] = (m - mu) / jnp.sqrt(v + EPS) * g2_ref[...] + be2_ref[...] + x


def _val_rec(x_val, partials, W3, b3, W4, g2, be2, *, block_rows=1000):
    nv, h = x_val.shape
    grid = nv // block_rows
    return pl.pallas_call(
        _val_rec_body,
        grid=(grid,),
        in_specs=[
            pl.BlockSpec((block_rows, h), lambda i: (i, 0)),
            pl.BlockSpec((NUM_SC, block_rows, h), lambda i: (0, i, 0)),
            pl.BlockSpec((h, h), lambda i: (0, 0)),
            pl.BlockSpec((1, h), lambda i: (0, 0)),
            pl.BlockSpec((h, h), lambda i: (0, 0)),
            pl.BlockSpec((1, h), lambda i: (0, 0)),
            pl.BlockSpec((1, h), lambda i: (0, 0)),
        ],
        out_specs=pl.BlockSpec((block_rows, h), lambda i: (i, 0)),
        out_shape=jax.ShapeDtypeStruct((nv, h), jnp.float32),
    )(x_val, partials, W3, b3.reshape(1, h), W4, g2.reshape(1, h), be2.reshape(1, h))


def kernel(x_val, r_cst, cst_edges, LE, PE, num_cst, num_val, W1, b1, W2,
           g1, be1, W3, b3, W4, g2, be2):
    nc, h = r_cst.shape
    nv = x_val.shape[0]
    m_cst = _cst_send(r_cst, W1, b1, W2, g1, be1)
    m_cst = m_cst.reshape(4 * nc, h)
    quantum = CHUNK * NUM_SUBCORES
    nvp = ((nv + quantum - 1) // quantum) * quantum
    partials = _edge_agg(m_cst, cst_edges.reshape(-1), LE, PE, nvp)
    return _val_rec(x_val, partials, W3, b3, W4, g2, be2)


# speedup vs baseline: 1.3665x; 1.1641x over previous
import functools

import jax
import jax.numpy as jnp
from jax import lax
from jax.experimental import pallas as pl
from jax.experimental.pallas import tpu as pltpu
from jax.experimental.pallas import tpu_sc as plsc

EPS = 1e-5

NUM_SC = 2
NUM_SUBCORES = 16
NUM_WORKERS = NUM_SC * NUM_SUBCORES
LANES = 16

CHUNK = 80
K = 2


def _cst_send_body(x_ref, w1_ref, b1_ref, w2_ref, g1_ref, be1_ref, o_ref):
    x = x_ref[...]
    h = lax.dot_general(x, w1_ref[...], (((1,), (1,)), ((), ())),
                        preferred_element_type=jnp.float32)
    h = jnp.maximum(h + b1_ref[...], 0.0)
    m = lax.dot_general(h, w2_ref[...], (((1,), (1,)), ((), ())),
                        preferred_element_type=jnp.float32)
    mu = jnp.mean(m, axis=-1, keepdims=True)
    v = jnp.mean((m - mu) ** 2, axis=-1, keepdims=True)
    o_ref[...] = (m - mu) / jnp.sqrt(v + EPS) * g1_ref[...] + be1_ref[...]


def _cst_send(r_cst, W1, b1, W2, g1, be1, *, block_rows=1000):
    nc, h = r_cst.shape
    h4 = W2.shape[0]
    grid = nc // block_rows
    return pl.pallas_call(
        _cst_send_body,
        grid=(grid,),
        in_specs=[
            pl.BlockSpec((block_rows, h), lambda i: (i, 0)),
            pl.BlockSpec((h, h), lambda i: (0, 0)),
            pl.BlockSpec((1, h), lambda i: (0, 0)),
            pl.BlockSpec((h4, h), lambda i: (0, 0)),
            pl.BlockSpec((1, h4), lambda i: (0, 0)),
            pl.BlockSpec((1, h4), lambda i: (0, 0)),
        ],
        out_specs=pl.BlockSpec((block_rows, h4), lambda i: (i, 0)),
        out_shape=jax.ShapeDtypeStruct((nc, h4), jnp.float32),
    )(r_cst, W1, b1.reshape(1, h), W2, g1.reshape(1, h4), be1.reshape(1, h4))


def _edge_agg(m_cst, ce_flat, le, pe, nvp):
    h = m_cst.shape[1]
    e = le.shape[0]
    ept = e // NUM_WORKERS
    rows = ept // CHUNK
    groups = rows // K
    tail = rows - K * groups
    rows_per_tile = nvp // NUM_SUBCORES
    zrep = rows_per_tile // CHUNK
    mesh = plsc.VectorSubcoreMesh(
        core_axis_name="c", subcore_axis_name="s",
        num_cores=NUM_SC, num_subcores=NUM_SUBCORES)

    IB = 2 * K

    @functools.partial(
        pl.kernel,
        out_type=jax.ShapeDtypeStruct((NUM_SC, nvp, h), jnp.float32),
        mesh=mesh,
        scratch_types=[
            [pltpu.VMEM((4, CHUNK), jnp.int32) for _ in range(IB)],
            [pltpu.VMEM((CHUNK, h), jnp.float32) for _ in range(K)],
            pltpu.VMEM_SHARED((nvp, h), jnp.float32),
            [pltpu.SemaphoreType.DMA for _ in range(K)],
            [pltpu.SemaphoreType.DMA for _ in range(IB)],
        ],
    )
    def edge_kernel(m_ref, ce_ref, le_ref, pe_ref, out_ref,
                    ibuf, msg, acc_ref, gsem, isem):
        c = lax.axis_index("c")
        s = lax.axis_index("s")
        wid = c * NUM_SUBCORES + s
        r0 = s * rows_per_tile
        base = wid * ept

        def zbody(t, carry):
            msg[0][t // (h // LANES),
                   pl.ds((t % (h // LANES)) * LANES, LANES)] = jnp.zeros(
                (LANES,), jnp.float32)
            return carry
        lax.fori_loop(0, CHUNK * h // LANES, zbody, 0)
        for z in range(zrep):
            pltpu.sync_copy(msg[0], acc_ref.at[pl.ds(r0 + z * CHUNK, CHUNK)])
        plsc.subcore_barrier()

        def idx_srcs(r):
            off = base + r * CHUNK
            return [ce_ref.at[pl.ds(off, CHUNK)],
                    le_ref.at[pl.ds(off, CHUNK)],
                    pe_ref.at[pl.ds(off, CHUNK)],
                    ce_ref.at[pl.ds(e + off, CHUNK)]]

        def issue_idx(r, v):
            for d, src in enumerate(idx_srcs(r)):
                pltpu.async_copy(src, ibuf[v].at[d], isem[v])

        def wait_idx(r, v):
            for d, src in enumerate(idx_srcs(r)):
                pltpu.make_async_copy(src, ibuf[v].at[d], isem[v]).wait()

        def oidx_and_gather(v, b):
            for cc in range(CHUNK // LANES):
                sl = pl.ds(cc * LANES, LANES)
                ibuf[v][0, sl] = (ibuf[v][0, sl] * 4 + ibuf[v][1, sl] * 2
                                  + ibuf[v][2, sl])
            return pltpu.async_copy(m_ref.at[ibuf[v].at[0]], msg[b], gsem[b])

        for r in range(min(IB, rows)):
            issue_idx(r, r)
        for b in range(min(K, rows)):
            wait_idx(b, b)
            oidx_and_gather(b, b)

        def ring_step(i, v):
            b = v % K
            pltpu.make_async_copy(m_ref.at[ibuf[v].at[0]], msg[b],
                                  gsem[b]).wait()
            pltpu.sync_copy(msg[b], acc_ref.at[ibuf[v].at[3]], add=True)

            @pl.when(i + IB < rows)
            def _():
                issue_idx(i + IB, v)

            @pl.when(i + K < rows)
            def _():
                w = (v + K) % IB
                wait_idx(i + K, w)
                oidx_and_gather(w, b)

        def rbody(i, carry):
            for v in range(IB):
                @pl.when(i % IB == v)
                def _():
                    ring_step(i, v)
            return carry

        lax.fori_loop(0, rows, rbody, 0)

        plsc.subcore_barrier()
        pltpu.sync_copy(acc_ref.at[pl.ds(r0, rows_per_tile)],
                        out_ref.at[c, pl.ds(r0, rows_per_tile)])

    return edge_kernel(m_cst, ce_flat, le, pe)


def _val_rec_body(x_ref, p_ref, w3_ref, b3_ref, w4_ref, g2_ref, be2_ref, o_ref):
    x = x_ref[...]
    z = x + p_ref[0] + p_ref[1]
    u = lax.dot_general(z, w3_ref[...], (((1,), (1,)), ((), ())),
                        preferred_element_type=jnp.float32)
    u = jnp.maximum(u + b3_ref[...], 0.0)
    m = lax.dot_general(u, w4_ref[...], (((1,), (1,)), ((), ())),
                        preferred_element_type=jnp.float32)
    mu = jnp.mean(m, axis=-1, keepdims=True)
    v = jnp.mean((m - mu) ** 2, axis=-1, keepdims=True)
    o_ref[...] = (m - mu) / jnp.sqrt(v + EPS) * g2_ref[...] + be2_ref[...] + x


def _val_rec(x_val, partials, W3, b3, W4, g2, be2, *, block_rows=1000):
    nv, h = x_val.shape
    grid = nv // block_rows
    return pl.pallas_call(
        _val_rec_body,
        grid=(grid,),
        in_specs=[
            pl.BlockSpec((block_rows, h), lambda i: (i, 0)),
            pl.BlockSpec((NUM_SC, block_rows, h), lambda i: (0, i, 0)),
            pl.BlockSpec((h, h), lambda i: (0, 0)),
            pl.BlockSpec((1, h), lambda i: (0, 0)),
            pl.BlockSpec((h, h), lambda i: (0, 0)),
            pl.BlockSpec((1, h), lambda i: (0, 0)),
            pl.BlockSpec((1, h), lambda i: (0, 0)),
        ],
        out_specs=pl.BlockSpec((block_rows, h), lambda i: (i, 0)),
        out_shape=jax.ShapeDtypeStruct((nv, h), jnp.float32),
    )(x_val, partials, W3, b3.reshape(1, h), W4, g2.reshape(1, h), be2.reshape(1, h))


def kernel(x_val, r_cst, cst_edges, LE, PE, num_cst, num_val, W1, b1, W2,
           g1, be1, W3, b3, W4, g2, be2):
    nc, h = r_cst.shape
    nv = x_val.shape[0]
    m_cst = _cst_send(r_cst, W1, b1, W2, g1, be1)
    m_cst = m_cst.reshape(4 * nc, h)
    quantum = CHUNK * NUM_SUBCORES
    nvp = ((nv + quantum - 1) // quantum) * quantum
    partials = _edge_agg(m_cst, cst_edges.reshape(-1), LE, PE, nvp)
    return _val_rec(x_val, partials, W3, b3, W4, g2, be2)


# speedup vs baseline: 1.4082x; 1.0305x over previous
import functools

import jax
import jax.numpy as jnp
from jax import lax
from jax.experimental import pallas as pl
from jax.experimental.pallas import tpu as pltpu
from jax.experimental.pallas import tpu_sc as plsc

EPS = 1e-5

NUM_SC = 2
NUM_SUBCORES = 16
NUM_WORKERS = NUM_SC * NUM_SUBCORES
LANES = 16

CHUNK = 80
K = 2


def _cst_send_body(x_ref, w1_ref, b1_ref, w2_ref, g1_ref, be1_ref, o_ref):
    x = x_ref[...]
    h = lax.dot_general(x, w1_ref[...], (((1,), (1,)), ((), ())),
                        preferred_element_type=jnp.float32)
    h = jnp.maximum(h + b1_ref[...], 0.0)
    m = lax.dot_general(h, w2_ref[...], (((1,), (1,)), ((), ())),
                        preferred_element_type=jnp.float32)
    mu = jnp.mean(m, axis=-1, keepdims=True)
    v = jnp.mean((m - mu) ** 2, axis=-1, keepdims=True)
    o_ref[...] = (m - mu) / jnp.sqrt(v + EPS) * g1_ref[...] + be1_ref[...]


def _cst_send(r_cst, W1, b1, W2, g1, be1, *, block_rows=2000):
    nc, h = r_cst.shape
    h4 = W2.shape[0]
    grid = nc // block_rows
    return pl.pallas_call(
        _cst_send_body,
        grid=(grid,),
        in_specs=[
            pl.BlockSpec((block_rows, h), lambda i: (i, 0)),
            pl.BlockSpec((h, h), lambda i: (0, 0)),
            pl.BlockSpec((1, h), lambda i: (0, 0)),
            pl.BlockSpec((h4, h), lambda i: (0, 0)),
            pl.BlockSpec((1, h4), lambda i: (0, 0)),
            pl.BlockSpec((1, h4), lambda i: (0, 0)),
        ],
        out_specs=pl.BlockSpec((block_rows, h4), lambda i: (i, 0)),
        out_shape=jax.ShapeDtypeStruct((nc, h4), jnp.float32),
    )(r_cst, W1, b1.reshape(1, h), W2, g1.reshape(1, h4), be1.reshape(1, h4))


def _edge_agg(m_cst, ce_flat, le, pe, nvp):
    h = m_cst.shape[1]
    e = le.shape[0]
    ept = e // NUM_WORKERS
    rows = ept // CHUNK
    groups = rows // K
    tail = rows - K * groups
    rows_per_tile = nvp // NUM_SUBCORES
    zrep = rows_per_tile // CHUNK
    mesh = plsc.VectorSubcoreMesh(
        core_axis_name="c", subcore_axis_name="s",
        num_cores=NUM_SC, num_subcores=NUM_SUBCORES)

    IB = 2 * K

    @functools.partial(
        pl.kernel,
        out_type=jax.ShapeDtypeStruct((NUM_SC, nvp, h), jnp.float32),
        mesh=mesh,
        scratch_types=[
            [pltpu.VMEM((4, CHUNK), jnp.int32) for _ in range(IB)],
            [pltpu.VMEM((CHUNK, h), jnp.float32) for _ in range(K)],
            pltpu.VMEM_SHARED((nvp, h), jnp.float32),
            [pltpu.SemaphoreType.DMA for _ in range(K)],
            [pltpu.SemaphoreType.DMA for _ in range(IB)],
        ],
    )
    def edge_kernel(m_ref, ce_ref, le_ref, pe_ref, out_ref,
                    ibuf, msg, acc_ref, gsem, isem):
        c = lax.axis_index("c")
        s = lax.axis_index("s")
        wid = c * NUM_SUBCORES + s
        r0 = s * rows_per_tile
        base = wid * ept

        def idx_srcs(r):
            off = base + r * CHUNK
            return [ce_ref.at[pl.ds(off, CHUNK)],
                    le_ref.at[pl.ds(off, CHUNK)],
                    pe_ref.at[pl.ds(off, CHUNK)],
                    ce_ref.at[pl.ds(e + off, CHUNK)]]

        def issue_idx(r, v):
            for d, src in enumerate(idx_srcs(r)):
                pltpu.async_copy(src, ibuf[v].at[d], isem[v])

        def wait_idx(r, v):
            for d, src in enumerate(idx_srcs(r)):
                pltpu.make_async_copy(src, ibuf[v].at[d], isem[v]).wait()

        for r in range(min(IB, rows)):
            issue_idx(r, r)

        def zbody(t, carry):
            msg[0][t // (h // LANES),
                   pl.ds((t % (h // LANES)) * LANES, LANES)] = jnp.zeros(
                (LANES,), jnp.float32)
            return carry
        lax.fori_loop(0, CHUNK * h // LANES, zbody, 0)
        for z in range(zrep):
            pltpu.sync_copy(msg[0], acc_ref.at[pl.ds(r0 + z * CHUNK, CHUNK)])
        plsc.subcore_barrier()

        def oidx_and_gather(v, b):
            for cc in range(CHUNK // LANES):
                sl = pl.ds(cc * LANES, LANES)
                ibuf[v][0, sl] = (ibuf[v][0, sl] * 4 + ibuf[v][1, sl] * 2
                                  + ibuf[v][2, sl])
            return pltpu.async_copy(m_ref.at[ibuf[v].at[0]], msg[b], gsem[b])

        for b in range(min(K, rows)):
            wait_idx(b, b)
            oidx_and_gather(b, b)

        def ring_step(i, v):
            b = v % K
            pltpu.make_async_copy(m_ref.at[ibuf[v].at[0]], msg[b],
                                  gsem[b]).wait()
            pltpu.sync_copy(msg[b], acc_ref.at[ibuf[v].at[3]], add=True)

            @pl.when(i + IB < rows)
            def _():
                issue_idx(i + IB, v)

            @pl.when(i + K < rows)
            def _():
                w = (v + K) % IB
                wait_idx(i + K, w)
                oidx_and_gather(w, b)

        def rbody(i, carry):
            for v in range(IB):
                @pl.when(i % IB == v)
                def _():
                    ring_step(i, v)
            return carry

        lax.fori_loop(0, rows, rbody, 0)

        plsc.subcore_barrier()
        pltpu.sync_copy(acc_ref.at[pl.ds(r0, rows_per_tile)],
                        out_ref.at[c, pl.ds(r0, rows_per_tile)])

    return edge_kernel(m_cst, ce_flat, le, pe)


def _val_rec_body(x_ref, p_ref, w3_ref, b3_ref, w4_ref, g2_ref, be2_ref, o_ref):
    x = x_ref[...]
    z = x + p_ref[0] + p_ref[1]
    u = lax.dot_general(z, w3_ref[...], (((1,), (1,)), ((), ())),
                        preferred_element_type=jnp.float32)
    u = jnp.maximum(u + b3_ref[...], 0.0)
    m = lax.dot_general(u, w4_ref[...], (((1,), (1,)), ((), ())),
                        preferred_element_type=jnp.float32)
    mu = jnp.mean(m, axis=-1, keepdims=True)
    v = jnp.mean((m - mu) ** 2, axis=-1, keepdims=True)
    o_ref[...] = (m - mu) / jnp.sqrt(v + EPS) * g2_ref[...] + be2_ref[...] + x


def _val_rec(x_val, partials, W3, b3, W4, g2, be2, *, block_rows=2000):
    nv, h = x_val.shape
    grid = nv // block_rows
    return pl.pallas_call(
        _val_rec_body,
        grid=(grid,),
        in_specs=[
            pl.BlockSpec((block_rows, h), lambda i: (i, 0)),
            pl.BlockSpec((NUM_SC, block_rows, h), lambda i: (0, i, 0)),
            pl.BlockSpec((h, h), lambda i: (0, 0)),
            pl.BlockSpec((1, h), lambda i: (0, 0)),
            pl.BlockSpec((h, h), lambda i: (0, 0)),
            pl.BlockSpec((1, h), lambda i: (0, 0)),
            pl.BlockSpec((1, h), lambda i: (0, 0)),
        ],
        out_specs=pl.BlockSpec((block_rows, h), lambda i: (i, 0)),
        out_shape=jax.ShapeDtypeStruct((nv, h), jnp.float32),
    )(x_val, partials, W3, b3.reshape(1, h), W4, g2.reshape(1, h), be2.reshape(1, h))


def kernel(x_val, r_cst, cst_edges, LE, PE, num_cst, num_val, W1, b1, W2,
           g1, be1, W3, b3, W4, g2, be2):
    nc, h = r_cst.shape
    nv = x_val.shape[0]
    m_cst = _cst_send(r_cst, W1, b1, W2, g1, be1)
    m_cst = m_cst.reshape(4 * nc, h)
    quantum = CHUNK * NUM_SUBCORES
    nvp = ((nv + quantum - 1) // quantum) * quantum
    partials = _edge_agg(m_cst, cst_edges.reshape(-1), LE, PE, nvp)
    return _val_rec(x_val, partials, W3, b3, W4, g2, be2)


# speedup vs baseline: 1.4106x; 1.0017x over previous
import functools

import jax
import jax.numpy as jnp
from jax import lax
from jax.experimental import pallas as pl
from jax.experimental.pallas import tpu as pltpu
from jax.experimental.pallas import tpu_sc as plsc

EPS = 1e-5

NUM_SC = 2
NUM_SUBCORES = 16
NUM_WORKERS = NUM_SC * NUM_SUBCORES
LANES = 16

CHUNK = 80
K = 2


def _cst_send_body(x_ref, w1_ref, b1_ref, w2_ref, g1_ref, be1_ref, o_ref):
    x = x_ref[...]
    h = lax.dot_general(x, w1_ref[...], (((1,), (1,)), ((), ())),
                        preferred_element_type=jnp.float32)
    h = jnp.maximum(h + b1_ref[...], 0.0)
    m = lax.dot_general(h, w2_ref[...], (((1,), (1,)), ((), ())),
                        preferred_element_type=jnp.float32)
    mu = jnp.mean(m, axis=-1, keepdims=True)
    v = jnp.mean((m - mu) ** 2, axis=-1, keepdims=True)
    o_ref[...] = (m - mu) / jnp.sqrt(v + EPS) * g1_ref[...] + be1_ref[...]


def _cst_send(r_cst, W1, b1, W2, g1, be1, *, block_rows=2000):
    nc, h = r_cst.shape
    h4 = W2.shape[0]
    grid = nc // block_rows
    return pl.pallas_call(
        _cst_send_body,
        grid=(grid,),
        in_specs=[
            pl.BlockSpec((block_rows, h), lambda i: (i, 0)),
            pl.BlockSpec((h, h), lambda i: (0, 0)),
            pl.BlockSpec((1, h), lambda i: (0, 0)),
            pl.BlockSpec((h4, h), lambda i: (0, 0)),
            pl.BlockSpec((1, h4), lambda i: (0, 0)),
            pl.BlockSpec((1, h4), lambda i: (0, 0)),
        ],
        out_specs=pl.BlockSpec((block_rows, h4), lambda i: (i, 0)),
        out_shape=jax.ShapeDtypeStruct((nc, h4), jnp.float32),
    )(r_cst, W1, b1.reshape(1, h), W2, g1.reshape(1, h4), be1.reshape(1, h4))


def _edge_agg(m_cst, ce_flat, le, pe, nvp):
    h = m_cst.shape[1]
    e = le.shape[0]
    ept = e // NUM_WORKERS
    rows = ept // CHUNK
    rows_per_tile = nvp // NUM_SUBCORES
    zrep = rows_per_tile // CHUNK
    mesh = plsc.VectorSubcoreMesh(
        core_axis_name="c", subcore_axis_name="s",
        num_cores=NUM_SC, num_subcores=NUM_SUBCORES)

    IB = 2 * K

    @functools.partial(
        pl.kernel,
        out_type=jax.ShapeDtypeStruct((NUM_SC, nvp, h), jnp.float32),
        mesh=mesh,
        scratch_types=[
            [pltpu.VMEM((4, CHUNK), jnp.int32) for _ in range(IB)],
            [pltpu.VMEM((CHUNK, h), jnp.float32) for _ in range(K)],
            pltpu.VMEM_SHARED((nvp, h), jnp.float32),
            [pltpu.SemaphoreType.DMA for _ in range(K)],
            [pltpu.SemaphoreType.DMA for _ in range(IB)],
        ],
    )
    def edge_kernel(m_ref, ce_ref, le_ref, pe_ref, out_ref,
                    ibuf, msg, acc_ref, gsem, isem):
        c = lax.axis_index("c")
        s = lax.axis_index("s")
        wid = c * NUM_SUBCORES + s
        r0 = s * rows_per_tile
        base = wid * ept

        def idx_srcs(r):
            off = base + r * CHUNK
            return [ce_ref.at[pl.ds(off, CHUNK)],
                    le_ref.at[pl.ds(off, CHUNK)],
                    pe_ref.at[pl.ds(off, CHUNK)],
                    ce_ref.at[pl.ds(e + off, CHUNK)]]

        def issue_idx(r, v):
            for d, src in enumerate(idx_srcs(r)):
                pltpu.async_copy(src, ibuf[v].at[d], isem[v])

        def wait_idx(r, v):
            for d, src in enumerate(idx_srcs(r)):
                pltpu.make_async_copy(src, ibuf[v].at[d], isem[v]).wait()

        for r in range(min(IB, rows)):
            issue_idx(r, r)

        def zbody(t, carry):
            msg[0][t // (h // LANES),
                   pl.ds((t % (h // LANES)) * LANES, LANES)] = jnp.zeros(
                (LANES,), jnp.float32)
            return carry
        lax.fori_loop(0, CHUNK * h // LANES, zbody, 0)
        for z in range(zrep):
            pltpu.sync_copy(msg[0], acc_ref.at[pl.ds(r0 + z * CHUNK, CHUNK)])
        plsc.subcore_barrier()

        def oidx_and_gather(v, b):
            for cc in range(CHUNK // LANES):
                sl = pl.ds(cc * LANES, LANES)
                ibuf[v][0, sl] = (ibuf[v][0, sl] * 4 + ibuf[v][1, sl] * 2
                                  + ibuf[v][2, sl])
            return pltpu.async_copy(m_ref.at[ibuf[v].at[0]], msg[b], gsem[b])

        for b in range(min(K, rows)):
            wait_idx(b, b)
            oidx_and_gather(b, b)

        def ring_step(i, v):
            b = v % K
            pltpu.make_async_copy(m_ref.at[ibuf[v].at[0]], msg[b],
                                  gsem[b]).wait()
            pltpu.sync_copy(msg[b], acc_ref.at[ibuf[v].at[3]], add=True)

            @pl.when(i + IB < rows)
            def _():
                issue_idx(i + IB, v)

            @pl.when(i + K < rows)
            def _():
                w = (v + K) % IB
                wait_idx(i + K, w)
                oidx_and_gather(w, b)

        def rbody(i, carry):
            for v in range(IB):
                @pl.when(i % IB == v)
                def _():
                    ring_step(i, v)
            return carry

        lax.fori_loop(0, rows, rbody, 0)

        plsc.subcore_barrier()
        pltpu.sync_copy(acc_ref.at[pl.ds(r0, rows_per_tile)],
                        out_ref.at[c, pl.ds(r0, rows_per_tile)])

    return edge_kernel(m_cst, ce_flat, le, pe)


def _val_rec_body(x_ref, p_ref, w3_ref, b3_ref, w4_ref, g2_ref, be2_ref, o_ref):
    x = x_ref[...]
    z = x + p_ref[0] + p_ref[1]
    u = lax.dot_general(z, w3_ref[...], (((1,), (1,)), ((), ())),
                        preferred_element_type=jnp.float32)
    u = jnp.maximum(u + b3_ref[...], 0.0)
    m = lax.dot_general(u, w4_ref[...], (((1,), (1,)), ((), ())),
                        preferred_element_type=jnp.float32)
    mu = jnp.mean(m, axis=-1, keepdims=True)
    v = jnp.mean((m - mu) ** 2, axis=-1, keepdims=True)
    o_ref[...] = (m - mu) / jnp.sqrt(v + EPS) * g2_ref[...] + be2_ref[...] + x


def _val_rec(x_val, partials, W3, b3, W4, g2, be2, *, block_rows=2000):
    nv, h = x_val.shape
    grid = nv // block_rows
    return pl.pallas_call(
        _val_rec_body,
        grid=(grid,),
        in_specs=[
            pl.BlockSpec((block_rows, h), lambda i: (i, 0)),
            pl.BlockSpec((NUM_SC, block_rows, h), lambda i: (0, i, 0)),
            pl.BlockSpec((h, h), lambda i: (0, 0)),
            pl.BlockSpec((1, h), lambda i: (0, 0)),
            pl.BlockSpec((h, h), lambda i: (0, 0)),
            pl.BlockSpec((1, h), lambda i: (0, 0)),
            pl.BlockSpec((1, h), lambda i: (0, 0)),
        ],
        out_specs=pl.BlockSpec((block_rows, h), lambda i: (i, 0)),
        out_shape=jax.ShapeDtypeStruct((nv, h), jnp.float32),
    )(x_val, partials, W3, b3.reshape(1, h), W4, g2.reshape(1, h), be2.reshape(1, h))


def kernel(x_val, r_cst, cst_edges, LE, PE, num_cst, num_val, W1, b1, W2,
           g1, be1, W3, b3, W4, g2, be2):
    nc, h = r_cst.shape
    nv = x_val.shape[0]
    m_cst = _cst_send(r_cst, W1, b1, W2, g1, be1)
    m_cst = m_cst.reshape(4 * nc, h)
    quantum = CHUNK * NUM_SUBCORES
    nvp = ((nv + quantum - 1) // quantum) * quantum
    partials = _edge_agg(m_cst, cst_edges.reshape(-1), LE, PE, nvp)
    return _val_rec(x_val, partials, W3, b3, W4, g2, be2)
